# baseline (speedup 1.0000x reference)
import functools

import jax
import jax.numpy as jnp
from jax import lax
from jax.experimental import pallas as pl
from jax.experimental.pallas import tpu as pltpu
from jax.experimental.pallas import tpu_sc as plsc

_ROWS = 128
_VOCAB = 100000
_STRIDE = 1000
_NUM_ALLOWED = 64
_W = 16000
_NEG_INF = float("-inf")
_LANES = 16


def _sc_gather(scores):
    mesh = plsc.VectorSubcoreMesh(core_axis_name="c", subcore_axis_name="s")

    @functools.partial(
        pl.kernel,
        out_type=jax.ShapeDtypeStruct((_NUM_ALLOWED * _ROWS,), jnp.float32),
        mesh=mesh,
        scratch_types=[
            pltpu.VMEM((_ROWS, 128), jnp.float32),
            pltpu.VMEM((_ROWS,), jnp.float32),
        ],
        compiler_params=pltpu.CompilerParams(needs_layout_passes=False),
    )
    def k(scores_hbm, out_hbm, win_ref, col_ref):
        wid = lax.axis_index("c") * 16 + lax.axis_index("s")
        for t in range(2):
            c = wid * 2 + t
            start = c * _STRIDE
            win = pl.multiple_of((start // 128) * 128, 128)
            lane = start - win
            pltpu.sync_copy(scores_hbm.at[:, pl.ds(win, 128)], win_ref)
            lane_vec = jnp.full((_LANES,), lane, jnp.int32)
            for b in range(_ROWS // _LANES):
                rows = lax.iota(jnp.int32, _LANES) + b * _LANES
                vals = plsc.load_gather(win_ref, [rows, lane_vec])
                col_ref[pl.ds(b * _LANES, _LANES)] = vals
            off = pl.multiple_of(c * _ROWS, 8)
            pltpu.sync_copy(col_ref, out_hbm.at[pl.ds(off, _ROWS)])

    return k(scores)


def _tc_fill_insert_body(g_ref, o_ref):
    j = pl.program_id(0)
    o_ref[...] = jnp.full(o_ref.shape, _NEG_INF, jnp.float32)

    per_blk = _W // _STRIDE
    for jj in range((_NUM_ALLOWED * _STRIDE) // _W):

        @pl.when(j == jj)
        def _(jj=jj):
            for k in range(per_blk):
                c = jj * per_blk + k
                o_ref[:, k * _STRIDE : k * _STRIDE + 1] = g_ref[:, c : c + 1]


def _tc_fill_insert(gathered):
    return pl.pallas_call(
        _tc_fill_insert_body,
        grid=(pl.cdiv(_VOCAB, _W),),
        in_specs=[pl.BlockSpec((_ROWS, _NUM_ALLOWED), lambda j: (0, 0))],
        out_specs=pl.BlockSpec((_ROWS, _W), lambda j: (0, j)),
        out_shape=jax.ShapeDtypeStruct((_ROWS, _VOCAB), jnp.float32),
    )(gathered)


def kernel(input_ids, scores):
    del input_ids
    gathered_rows = _sc_gather(scores)
    gathered = gathered_rows.reshape(_NUM_ALLOWED, _ROWS).T
    return _tc_fill_insert(gathered)


# speedup vs baseline: 1.1338x; 1.1338x over previous
import jax
import jax.numpy as jnp
from jax.experimental import pallas as pl
from jax.experimental.pallas import tpu as pltpu

_ROWS = 128
_VOCAB = 100000
_STRIDE = 1000
_NUM_ALLOWED = 64
_W = 16000
_PER_BLK = _W // _STRIDE
_N_INSERT_BLOCKS = (_NUM_ALLOWED * _STRIDE) // _W
_NEG_INF = float("-inf")


def _body(scores_ref, o_ref, win_ref, sem):
    j = pl.program_id(0)

    for jj in range(_N_INSERT_BLOCKS):

        @pl.when(j == jj)
        def _(jj=jj):
            copies = []
            for k in range(_PER_BLK):
                col = (jj * _PER_BLK + k) * _STRIDE
                win = (col // 128) * 128
                cp = pltpu.make_async_copy(
                    scores_ref.at[:, pl.ds(win, 128)], win_ref.at[k], sem
                )
                cp.start()
                copies.append(cp)
            o_ref[...] = jnp.full(o_ref.shape, _NEG_INF, jnp.float32)
            for k in range(_PER_BLK):
                copies[k].wait()
                col = (jj * _PER_BLK + k) * _STRIDE
                r = col % 128
                o_ref[:, k * _STRIDE : k * _STRIDE + 1] = win_ref[k][:, r : r + 1]

    @pl.when(j >= _N_INSERT_BLOCKS)
    def _():
        o_ref[...] = jnp.full(o_ref.shape, _NEG_INF, jnp.float32)


def kernel(input_ids, scores):
    del input_ids
    return pl.pallas_call(
        _body,
        grid=(pl.cdiv(_VOCAB, _W),),
        in_specs=[pl.BlockSpec(memory_space=pl.ANY)],
        out_specs=pl.BlockSpec((_ROWS, _W), lambda j: (0, j)),
        out_shape=jax.ShapeDtypeStruct((_ROWS, _VOCAB), jnp.float32),
        scratch_shapes=[
            pltpu.VMEM((_PER_BLK, _ROWS, 128), jnp.float32),
            pltpu.SemaphoreType.DMA,
        ],
    )(scores)


# speedup vs baseline: 7.4503x; 6.5711x over previous
import jax
import jax.numpy as jnp
from jax.experimental import pallas as pl
from jax.experimental.pallas import tpu as pltpu

_VOCAB = 100000
_ROWS = 128
_STRIDE = 1000
_NUM_ALLOWED = 64
_CHUNK = 10000
_NCHUNK = _VOCAB // _CHUNK
_NEG_INF = float("-inf")


def _body(s_ref, o_ref, const_ref, fill_sems, ins_sem):
    const_ref[...] = jnp.full(const_ref.shape, _NEG_INF, jnp.float32)

    fills = []
    for b in range(_NCHUNK):
        cp = pltpu.make_async_copy(
            const_ref, o_ref.at[pl.ds(b * _CHUNK, _CHUNK), :], fill_sems.at[b]
        )
        cp.start()
        fills.append(cp)

    inserts = []
    for b in range(_NCHUNK):
        fills[b].wait()
        lo = -(-b * _CHUNK // _STRIDE)
        hi = min(_NUM_ALLOWED, ((b + 1) * _CHUNK - 1) // _STRIDE + 1)
        for c in range(lo, hi):
            r = c * _STRIDE
            cp = pltpu.make_async_copy(
                s_ref.at[pl.ds(r, 1), :], o_ref.at[pl.ds(r, 1), :], ins_sem
            )
            cp.start()
            inserts.append(cp)

    for cp in inserts:
        cp.wait()


def kernel(input_ids, scores):
    del input_ids
    scores_t = scores.T
    out_t = pl.pallas_call(
        _body,
        grid=(1,),
        in_specs=[pl.BlockSpec(memory_space=pl.ANY)],
        out_specs=pl.BlockSpec(memory_space=pl.ANY),
        out_shape=jax.ShapeDtypeStruct((_VOCAB, _ROWS), jnp.float32),
        scratch_shapes=[
            pltpu.VMEM((_CHUNK, _ROWS), jnp.float32),
            pltpu.SemaphoreType.DMA((_NCHUNK,)),
            pltpu.SemaphoreType.DMA,
        ],
    )(scores_t)
    return out_t.T
